# Initial kernel scaffold; baseline (speedup 1.0000x reference)
import jax
import jax.numpy as jnp
from jax.experimental import pallas as pl


def kernel(x, Wq, bq, Wk, bk, Wv, bv, Wp, bp):
    raise NotImplementedError("write your pallas kernel here")


# speedup vs baseline: 2.2413x; 2.2413x over previous
import math

import jax
import jax.numpy as jnp
import numpy as np
from jax.experimental import pallas as pl
from jax.experimental.pallas import tpu as pltpu

H = 16
B = 2
T = 2048
C = 1024
D = C // H
U = 40
BT = 256
TT = T // BT


def _build_cnt_i8() -> np.ndarray:
    with jax.default_device(jax.devices("cpu")[0]):
        idx = jax.random.randint(jax.random.key(42), (T, U), 0, T)
        idx_np = np.asarray(idx)
    cnt = np.zeros((T, T), dtype=np.int8)
    np.add.at(cnt, (np.arange(T)[:, None], idx_np), 1)
    return cnt


_CNT_I8 = _build_cnt_i8()
_TRI = np.tril(np.ones((BT, BT), dtype=np.float32))


def _proj_kernel(x_ref, wq_ref, wk_ref, wv_ref, bq_ref, bk_ref, bv_ref,
                 q_ref, k_ref, v_ref):
    xt = x_ref[0]
    qt = jnp.dot(xt, wq_ref[...], preferred_element_type=jnp.float32) + bq_ref[...]
    kt = jnp.dot(xt, wk_ref[...], preferred_element_type=jnp.float32) + bk_ref[...]
    vt = jnp.dot(xt, wv_ref[...], preferred_element_type=jnp.float32) + bv_ref[...]
    for hh in range(H):
        q_ref[0, hh] = qt[:, hh * D:(hh + 1) * D]
        k_ref[0, hh] = kt[:, hh * D:(hh + 1) * D]
        v_ref[0, hh] = vt[:, hh * D:(hh + 1) * D]


def _attn_kernel(q_ref, k_ref, v_ref, cnt_ref, updfull_ref, sel_ref,
                 m_s, qsel_s, midx_s, idx_smem):
    k_all = k_ref[0, 0]
    for tq in range(TT):
        qt = q_ref[0, 0, tq * BT:(tq + 1) * BT, :]
        s = jax.lax.dot_general(qt, k_all, (((1,), (1,)), ((), ())),
                                preferred_element_type=jnp.float32)
        cntt = cnt_ref[tq * BT:(tq + 1) * BT, :].astype(jnp.float32)
        mx = jnp.max(jnp.where(cntt > 0.0, s, -jnp.inf), axis=1, keepdims=True)
        sm = jnp.sum(s * cntt, axis=1, keepdims=True) * (1.0 / T)
        m_s[tq * BT:(tq + 1) * BT, :] = mx - sm

    row_iota = jax.lax.broadcasted_iota(jnp.int32, (T, 1), 0)

    def topk_body(i, carry):
        m = m_s[...]
        gmax = jnp.max(m)
        idx = jnp.min(jnp.where(m == gmax, row_iota, 2 * T))
        idx_smem[i] = idx
        midx_s[pl.ds(i, 1), :] = jnp.reshape(idx.astype(jnp.float32), (1, 1))
        qsel_s[pl.ds(i, 1), :] = q_ref[0, 0, pl.ds(idx, 1), :]
        m_s[pl.ds(idx, 1), :] = jnp.full((1, 1), -jnp.inf, jnp.float32)
        return carry

    jax.lax.fori_loop(0, U, topk_body, 0)

    qsel = qsel_s[...]
    scores = jax.lax.dot_general(qsel, k_all, (((1,), (1,)), ((), ())),
                                 preferred_element_type=jnp.float32)
    scores = scores * (1.0 / math.sqrt(D))
    col = jax.lax.broadcasted_iota(jnp.int32, (U, T), 1).astype(jnp.float32)
    scores = jnp.where(col > midx_s[...], -jnp.inf, scores)
    scores = scores - jnp.max(scores, axis=1, keepdims=True)
    p = jnp.exp(scores)
    attn = p / jnp.sum(p, axis=1, keepdims=True)
    upd = jax.lax.dot_general(attn, v_ref[0, 0], (((1,), (0,)), ((), ())),
                              preferred_element_type=jnp.float32)

    updfull_ref[0, 0] = jnp.zeros((T, D), jnp.float32)
    sel_ref[0, 0] = jnp.zeros((T, 1), jnp.float32)
    qsel_s[...] = upd

    def scat_body(i, carry):
        t = idx_smem[i]
        updfull_ref[0, 0, pl.ds(t, 1), :] = qsel_s[pl.ds(i, 1), :]
        sel_ref[0, 0, pl.ds(t, 1), :] = jnp.ones((1, 1), jnp.float32)
        return carry

    jax.lax.fori_loop(0, U, scat_body, 0)


def _ctx_kernel(v_ref, updfull_ref, sel_ref, tri_ref, wp_ref, bp_ref,
                out_ref, carry_s):
    t = pl.program_id(1)
    h = pl.program_id(2)

    @pl.when(t == 0)
    def _init_carry():
        carry_s[pl.ds(h, 1), :] = jnp.zeros((1, D), jnp.float32)

    vt = v_ref[0, 0]
    ctx = jnp.dot(tri_ref[...], vt, preferred_element_type=jnp.float32)
    ctx = ctx + carry_s[pl.ds(h, 1), :]
    carry_s[pl.ds(h, 1), :] = ctx[BT - 1:BT, :]
    ctx = jnp.where(sel_ref[0, 0] > 0.0, updfull_ref[0, 0], ctx)
    wp_slice = wp_ref[pl.ds(h * D, D), :]
    contrib = jnp.dot(ctx, wp_slice, preferred_element_type=jnp.float32)

    @pl.when(h == 0)
    def _first():
        out_ref[0] = bp_ref[...] + contrib

    @pl.when(h > 0)
    def _rest():
        out_ref[0] += contrib


def kernel(x, Wq, bq, Wk, bk, Wv, bv, Wp, bp):
    bq2 = bq.reshape(1, C)
    bk2 = bk.reshape(1, C)
    bv2 = bv.reshape(1, C)
    bp2 = bp.reshape(1, C)
    cnt = jnp.asarray(_CNT_I8)
    tri = jnp.asarray(_TRI)
    f32 = jnp.float32

    q, k, v = pl.pallas_call(
        _proj_kernel,
        grid=(B, TT),
        in_specs=[
            pl.BlockSpec((1, BT, C), lambda b, t: (b, t, 0)),
            pl.BlockSpec((C, C), lambda b, t: (0, 0)),
            pl.BlockSpec((C, C), lambda b, t: (0, 0)),
            pl.BlockSpec((C, C), lambda b, t: (0, 0)),
            pl.BlockSpec((1, C), lambda b, t: (0, 0)),
            pl.BlockSpec((1, C), lambda b, t: (0, 0)),
            pl.BlockSpec((1, C), lambda b, t: (0, 0)),
        ],
        out_specs=[
            pl.BlockSpec((1, H, BT, D), lambda b, t: (b, 0, t, 0)),
            pl.BlockSpec((1, H, BT, D), lambda b, t: (b, 0, t, 0)),
            pl.BlockSpec((1, H, BT, D), lambda b, t: (b, 0, t, 0)),
        ],
        out_shape=[
            jax.ShapeDtypeStruct((B, H, T, D), f32),
            jax.ShapeDtypeStruct((B, H, T, D), f32),
            jax.ShapeDtypeStruct((B, H, T, D), f32),
        ],
        compiler_params=pltpu.CompilerParams(
            dimension_semantics=("parallel", "parallel")),
    )(x, Wq, Wk, Wv, bq2, bk2, bv2)

    updfull, sel = pl.pallas_call(
        _attn_kernel,
        grid=(B, H),
        in_specs=[
            pl.BlockSpec((1, 1, T, D), lambda b, h: (b, h, 0, 0)),
            pl.BlockSpec((1, 1, T, D), lambda b, h: (b, h, 0, 0)),
            pl.BlockSpec((1, 1, T, D), lambda b, h: (b, h, 0, 0)),
            pl.BlockSpec((T, T), lambda b, h: (0, 0)),
        ],
        out_specs=[
            pl.BlockSpec((1, 1, T, D), lambda b, h: (b, h, 0, 0)),
            pl.BlockSpec((1, 1, T, 1), lambda b, h: (b, h, 0, 0)),
        ],
        out_shape=[
            jax.ShapeDtypeStruct((B, H, T, D), f32),
            jax.ShapeDtypeStruct((B, H, T, 1), f32),
        ],
        scratch_shapes=[
            pltpu.VMEM((T, 1), f32),
            pltpu.VMEM((U, D), f32),
            pltpu.VMEM((U, 1), f32),
            pltpu.SMEM((U,), jnp.int32),
        ],
        compiler_params=pltpu.CompilerParams(
            dimension_semantics=("parallel", "parallel")),
    )(q, k, v, cnt)

    out = pl.pallas_call(
        _ctx_kernel,
        grid=(B, TT, H),
        in_specs=[
            pl.BlockSpec((1, 1, BT, D), lambda b, t, h: (b, h, t, 0)),
            pl.BlockSpec((1, 1, BT, D), lambda b, t, h: (b, h, t, 0)),
            pl.BlockSpec((1, 1, BT, 1), lambda b, t, h: (b, h, t, 0)),
            pl.BlockSpec((BT, BT), lambda b, t, h: (0, 0)),
            pl.BlockSpec((C, C), lambda b, t, h: (0, 0)),
            pl.BlockSpec((1, C), lambda b, t, h: (0, 0)),
        ],
        out_specs=pl.BlockSpec((1, BT, C), lambda b, t, h: (b, t, 0)),
        out_shape=jax.ShapeDtypeStruct((B, T, C), f32),
        scratch_shapes=[pltpu.VMEM((H, D), f32)],
        compiler_params=pltpu.CompilerParams(
            dimension_semantics=("parallel", "arbitrary", "arbitrary")),
    )(v, updfull, sel, tri, Wp, bp2)

    return out
